# Initial kernel scaffold; baseline (speedup 1.0000x reference)
import jax
import jax.numpy as jnp
from jax.experimental import pallas as pl


def kernel(scores, L_sym):
    raise NotImplementedError("write your pallas kernel here")


# speedup vs baseline: 1.1970x; 1.1970x over previous
import functools

import jax
import jax.numpy as jnp
from jax.experimental import pallas as pl
from jax.experimental.pallas import tpu as pltpu

_ITERS = 10
_K = 0
_TAU = 0.5
_EPS = 1e-12
_N = 4096
_M = 256
_BR = 512
_NB = _N // _BR

_DOT_DIMS = (((1,), (0,)), ((), ()))


def _soft(y):
    return y - _TAU * jnp.sign(y)


def _tree(total):
    tr = total[0:4, :] + total[4:8, :]
    tr = tr[0:2, :] + tr[2:4, :]
    tr = tr[0:1, :] + tr[1:2, :]
    return tr


def _chain(w2, t0, nt):
    acc = w2[8 * t0 : 8 * t0 + 8, :]
    for t in range(t0 + 1, t0 + nt):
        acc = acc + w2[8 * t : 8 * (t + 1), :]
    return acc


def _colsum_first(w2):
    return _tree(_chain(w2, 0, _N // 8))


def _colsum_rest(w2):
    acc_e = w2[0:8, :]
    acc_o = w2[8:16, :]
    for t in range(2, _N // 8):
        tile = w2[8 * t : 8 * (t + 1), :]
        if t % 2 == 0:
            acc_e = acc_e + tile
        else:
            acc_o = acc_o + tile
    return _tree(acc_e + acc_o)


def _normalize(y, first):
    w = _soft(y)
    s = _colsum_first(w * w) if first else _colsum_rest(w * w)
    norm = jnp.maximum(jnp.sqrt(s), _EPS)
    return w / norm


def _body(vin_ref, L_ref, out_ref, v_scr, y_scr):
    i = pl.program_id(0)
    j = pl.program_id(1)
    n_pallas = _ITERS - _K

    @pl.when(j == 0)
    def _prep():
        @pl.when(i == 0)
        def _():
            v_scr[...] = vin_ref[...]

        if _K == 0:
            @pl.when(i == 1)
            def _():
                v_scr[...] = _normalize(y_scr[...], True)

            @pl.when(i > 1)
            def _():
                v_scr[...] = _normalize(y_scr[...], False)
        else:
            @pl.when(i > 0)
            def _():
                v_scr[...] = _normalize(y_scr[...], False)

    def _dot(lhs, rhs):
        return jax.lax.dot_general(
            lhs,
            rhs,
            _DOT_DIMS,
            precision=jax.lax.Precision.DEFAULT,
            preferred_element_type=jnp.float32,
        )

    if _K == 0:
        @pl.when(i == 0)
        def _mm_first():
            y_scr[pl.ds(j * _BR, _BR), :] = _dot(
                L_ref[...].astype(jnp.bfloat16), v_scr[...].astype(jnp.bfloat16)
            )

        @pl.when(i > 0)
        def _mm_rest():
            y_scr[pl.ds(j * _BR, _BR), :] = _dot(L_ref[...], v_scr[...])
    else:
        y_scr[pl.ds(j * _BR, _BR), :] = _dot(L_ref[...], v_scr[...])

    @pl.when(jnp.logical_and(i == n_pallas - 1, j == _NB - 1))
    def _finish():
        out_ref[...] = _normalize(y_scr[...], False)


def _pallas_run(v, L_sym):
    return pl.pallas_call(
        _body,
        grid=(_ITERS - _K, _NB),
        in_specs=[
            pl.BlockSpec((_N, _M), lambda i, j: (0, 0)),
            pl.BlockSpec((_BR, _N), lambda i, j: (j, 0)),
        ],
        out_specs=pl.BlockSpec((_N, _M), lambda i, j: (0, 0)),
        out_shape=jax.ShapeDtypeStruct((_N, _M), jnp.float32),
        scratch_shapes=[
            pltpu.VMEM((_N, _M), jnp.float32),
            pltpu.VMEM((_N, _M), jnp.float32),
        ],
    )(v, L_sym)


@jax.jit
def kernel(scores, L_sym):
    v = scores
    for _ in range(_K):
        v = L_sym @ v
        v = v - _TAU * jnp.sign(v)
        norm = jnp.sqrt(jnp.sum(v * v, axis=0, keepdims=True))
        v = v / jnp.maximum(norm, 1e-12)
    return _pallas_run(v, L_sym)
